# Initial kernel scaffold; baseline (speedup 1.0000x reference)
import jax
import jax.numpy as jnp
from jax.experimental import pallas as pl


def kernel(x1_data, x1_batch_indices, x2_data, x2_batch_indices, g1_edge_index, g1_edge_attr, g1_batch_id, g2_edge_index, g2_edge_attr, g2_batch_id, W_rel, W_self):
    raise NotImplementedError("write your pallas kernel here")


# speedup vs baseline: 22.3736x; 22.3736x over previous
import functools

import jax
import jax.numpy as jnp
from jax import lax
from jax.experimental import pallas as pl
from jax.experimental.pallas import tpu as pltpu
import jax.experimental.pallas.tpu_sc as plsc

N = 10000
E = 320000
D = 128
R = 4
G = 64

NC = 2
NS = 16

EPT = E // NS
CH = 128
NFULL = EPT // CH
TAIL = EPT - NFULL * CH
RPT = 632

BN = 1000
PBN = 2000
NPB = N // PBN


def _transform_body(relu_in, x_ref, wr_ref, ws_ref, xt_ref, xs_ref):
    x = x_ref[0]
    if relu_in:
        x = jnp.maximum(x, 0.0)
    for r in range(R):
        xt_ref[0, r] = jnp.dot(x, wr_ref[r], preferred_element_type=jnp.float32)
    xs_ref[0] = jnp.dot(x, ws_ref[...], preferred_element_type=jnp.float32)


def _transform(x, w_rel, w_self, relu_in):
    return pl.pallas_call(
        functools.partial(_transform_body, relu_in),
        grid=(2, N // BN),
        in_specs=[
            pl.BlockSpec((1, BN, D), lambda g, b: (g, b, 0)),
            pl.BlockSpec((R, D, D), lambda g, b: (0, 0, 0)),
            pl.BlockSpec((D, D), lambda g, b: (0, 0)),
        ],
        out_specs=[
            pl.BlockSpec((1, R, BN, D), lambda g, b: (g, 0, b, 0)),
            pl.BlockSpec((1, BN, D), lambda g, b: (g, b, 0)),
        ],
        out_shape=[
            jax.ShapeDtypeStruct((2, R, N, D), jnp.float32),
            jax.ShapeDtypeStruct((2, N, D), jnp.float32),
        ],
    )(x, w_rel, w_self)


def _edge_agg_body(xt_hbm, gidx_hbm, dst_hbm, xself_hbm, out_hbm,
                   gv, dv, rv, gtv, dtv, rtv, acc, sem):
    c = lax.axis_index("c")
    s = lax.axis_index("s")

    r0 = pl.multiple_of(jnp.minimum(s * RPT, N - RPT), 8)
    pltpu.sync_copy(xself_hbm.at[c, pl.ds(r0, RPT)], acc.at[pl.ds(r0, RPT)])
    plsc.subcore_barrier()

    ebase = c * E + s * EPT

    def chunk(i, carry):
        off = pl.multiple_of(ebase + i * CH, 8)
        pltpu.sync_copy(gidx_hbm.at[pl.ds(off, CH)], gv)
        pltpu.sync_copy(dst_hbm.at[pl.ds(off, CH)], dv)
        pltpu.async_copy(xt_hbm.at[gv], rv, sem).wait()
        pltpu.sync_copy(rv, acc.at[dv], add=True)
        return carry

    lax.fori_loop(0, NFULL, chunk, 0)

    toff = pl.multiple_of(ebase + NFULL * CH, 8)
    pltpu.sync_copy(gidx_hbm.at[pl.ds(toff, TAIL)], gtv)
    pltpu.sync_copy(dst_hbm.at[pl.ds(toff, TAIL)], dtv)
    pltpu.async_copy(xt_hbm.at[gtv], rtv, sem).wait()
    pltpu.sync_copy(rtv, acc.at[dtv], add=True)

    plsc.subcore_barrier()
    pltpu.sync_copy(acc.at[pl.ds(r0, RPT)], out_hbm.at[c, pl.ds(r0, RPT)])


_edge_agg = pl.kernel(
    _edge_agg_body,
    out_type=jax.ShapeDtypeStruct((2, N, D), jnp.float32),
    mesh=plsc.VectorSubcoreMesh(core_axis_name="c", subcore_axis_name="s",
                                num_cores=NC, num_subcores=NS),
    scratch_types=[
        pltpu.VMEM((CH,), jnp.int32),
        pltpu.VMEM((CH,), jnp.int32),
        pltpu.VMEM((CH, D), jnp.float32),
        pltpu.VMEM((TAIL,), jnp.int32),
        pltpu.VMEM((TAIL,), jnp.int32),
        pltpu.VMEM((TAIL, D), jnp.float32),
        pltpu.VMEM_SHARED((N, D), jnp.float32),
        pltpu.SemaphoreType.DMA,
    ],
)


def _pool_body(agg_ref, bidx_ref, out_ref, sums_ref, cnts_ref):
    b = pl.program_id(0)

    @pl.when(b == 0)
    def _():
        sums_ref[...] = jnp.zeros_like(sums_ref)
        cnts_ref[...] = jnp.zeros_like(cnts_ref)

    for g in range(2):
        x = jnp.maximum(agg_ref[g], 0.0)
        bi = bidx_ref[g, 0, 0]
        onehot = (bi[None, :] == lax.broadcasted_iota(jnp.int32, (G, PBN), 0))
        onehot = onehot.astype(jnp.float32)
        sums_ref[g] += jnp.dot(onehot, x, preferred_element_type=jnp.float32)
        cnts_ref[g] += jnp.sum(onehot, axis=1, keepdims=True)

    @pl.when(b == NPB - 1)
    def _():
        v1 = sums_ref[0] / jnp.maximum(cnts_ref[0], 1.0)
        v2 = sums_ref[1] / jnp.maximum(cnts_ref[1], 1.0)
        out_ref[:, 0 * D:1 * D] = v1
        out_ref[:, 1 * D:2 * D] = v2
        out_ref[:, 2 * D:3 * D] = v1 - v2
        out_ref[:, 3 * D:4 * D] = v1 * v2


def _pool_concat(agg, bidx):
    return pl.pallas_call(
        _pool_body,
        grid=(NPB,),
        in_specs=[
            pl.BlockSpec((2, PBN, D), lambda b: (0, b, 0)),
            pl.BlockSpec((2, 1, 1, PBN), lambda b: (0, b, 0, 0)),
        ],
        out_specs=pl.BlockSpec((G, 4 * D), lambda b: (0, 0)),
        out_shape=jax.ShapeDtypeStruct((G, 4 * D), jnp.float32),
        scratch_shapes=[
            pltpu.VMEM((2, G, D), jnp.float32),
            pltpu.VMEM((2, G, 1), jnp.float32),
        ],
    )(agg, bidx)


def kernel(x1_data, x1_batch_indices, x2_data, x2_batch_indices,
           g1_edge_index, g1_edge_attr, g1_batch_id,
           g2_edge_index, g2_edge_attr, g2_batch_id,
           W_rel, W_self):
    x = jnp.stack([x1_data, x2_data])

    gidx = jnp.concatenate([
        g1_edge_attr.astype(jnp.int32) * N + g1_edge_index[0].astype(jnp.int32),
        R * N + g2_edge_attr.astype(jnp.int32) * N + g2_edge_index[0].astype(jnp.int32),
    ])
    dst = jnp.concatenate([g1_edge_index[1].astype(jnp.int32),
                           g2_edge_index[1].astype(jnp.int32)])

    for layer in range(2):
        xt, xself = _transform(x, W_rel, W_self, relu_in=(layer > 0))
        x = _edge_agg(xt.reshape(2 * R * N, D), gidx, dst, xself)

    bidx = jnp.stack([x1_batch_indices.astype(jnp.int32),
                      x2_batch_indices.astype(jnp.int32)]).reshape(2, NPB, 1, PBN)
    return _pool_concat(x, bidx)
